# baseline (speedup 1.0000x reference)
import jax
import jax.numpy as jnp
from jax import lax
from jax.experimental import pallas as pl
from jax.experimental.pallas import tpu as pltpu

_LN2 = 0.6931471805599453


def _fused_body(x_ref, f_ref, nbh_ref, mask_ref, win_ref, wf1_ref, bf1_ref,
                wf2_ref, bf2_ref, wout_ref, bout_ref, out_ref):
    nn, na = nbh_ref.shape[1], nbh_ref.shape[2]
    ng = f_ref.shape[1]
    rows = nn * na

    y = jnp.dot(x_ref[0], win_ref[...], preferred_element_type=jnp.float32)

    f = f_ref[0].reshape(ng, rows)
    h = lax.dot_general(f, wf1_ref[...], (((0,), (0,)), ((), ())),
                        preferred_element_type=jnp.float32) + bf1_ref[...]
    u = (jnp.log2(1.0 + jnp.exp(h)) - 1.0) * _LN2
    w = jnp.dot(u, wf2_ref[...], preferred_element_type=jnp.float32) + bf2_ref[...]

    nbh = jnp.where(mask_ref[0] != 0.0, nbh_ref[0], na)
    onehot = (lax.broadcasted_iota(jnp.int32, (nn, na, na), 2)
              == nbh[:, :, None]).astype(jnp.float32)
    y_g = jnp.dot(onehot.reshape(rows, na), y,
                  preferred_element_type=jnp.float32)

    agg = jnp.sum((w * y_g).reshape(nn, na, -1), axis=0)
    out_ref[0] = jnp.dot(agg, wout_ref[...],
                         preferred_element_type=jnp.float32) + bout_ref[...]


def kernel(x, r_ij, neighbors, pairwise_mask, f_ij, W_in2f, W_f1, b_f1,
           W_f2, b_f2, W_out, b_out):
    Nb, Na, nin = x.shape
    Nn = neighbors.shape[-1]
    ng = f_ij.shape[-1]
    nf = W_f1.shape[-1]
    nout = W_out.shape[-1]

    ft = jnp.transpose(f_ij, (0, 3, 2, 1))
    nbt = jnp.transpose(neighbors.astype(jnp.int32), (0, 2, 1))
    mt = jnp.transpose(pairwise_mask, (0, 2, 1))

    out = pl.pallas_call(
        _fused_body,
        grid=(Nb,),
        in_specs=[
            pl.BlockSpec((1, Na, nin), lambda b: (b, 0, 0)),
            pl.BlockSpec((1, ng, Nn, Na), lambda b: (b, 0, 0, 0)),
            pl.BlockSpec((1, Nn, Na), lambda b: (b, 0, 0)),
            pl.BlockSpec((1, Nn, Na), lambda b: (b, 0, 0)),
            pl.BlockSpec((nin, nf), lambda b: (0, 0)),
            pl.BlockSpec((ng, nf), lambda b: (0, 0)),
            pl.BlockSpec((1, nf), lambda b: (0, 0)),
            pl.BlockSpec((nf, nf), lambda b: (0, 0)),
            pl.BlockSpec((1, nf), lambda b: (0, 0)),
            pl.BlockSpec((nf, nout), lambda b: (0, 0)),
            pl.BlockSpec((1, nout), lambda b: (0, 0)),
        ],
        out_specs=pl.BlockSpec((1, Na, nout), lambda b: (b, 0, 0)),
        out_shape=jax.ShapeDtypeStruct((Nb, Na, nout), jnp.float32),
        compiler_params=pltpu.CompilerParams(
            dimension_semantics=("arbitrary",),
        ),
    )(x, ft, nbt, mt, W_in2f, W_f1, b_f1.reshape(1, -1), W_f2,
      b_f2.reshape(1, -1), W_out, b_out.reshape(1, -1))
    return out
